# baseline (speedup 1.0000x reference)
import functools

import jax
import jax.numpy as jnp
from jax import lax
from jax.experimental import pallas as pl
from jax.experimental.pallas import tpu as pltpu
from jax.experimental.pallas import tpu_sc as plsc

N_NODES = 100000
N_EDGES = 1600000
D = 32
L = 16
NC = 2
NS = 16
NW = NC * NS
SUB = 128
NSUB = 4
CHUNK = SUB * NSUB
NCHUNKS = N_EDGES // CHUNK
BASE_CHUNKS = NCHUNKS // NW
EXTRA = NCHUNKS - BASE_CHUNKS * NW
DUMP = 6256
DUMP_LAST = N_NODES - 15 * DUMP

_mesh = plsc.VectorSubcoreMesh(core_axis_name="c", subcore_axis_name="s")


def _edge_body(src2_h, dst2_h, eu_h, ev_h, g2b_h, zeros_h,
               ag_h, psum0_h, psum1_h,
               src_v, dst_v, u_v, v_v, ag_v, g2_v, sums_s,
               sem_idx, sem_g0, sem_g1, sem_g2, sem_g3, sem_s, sem_a):
    cid = lax.axis_index("c")
    sid = lax.axis_index("s")
    wid = sid * NC + cid
    gsems = (sem_g0, sem_g1, sem_g2, sem_g3)

    @pl.when(sid == 0)
    def _():
        pltpu.sync_copy(zeros_h, sums_s)

    pltpu.sync_copy(g2b_h, g2_v)
    plsc.subcore_barrier()

    nmine = jnp.where(wid < EXTRA, BASE_CHUNKS + 1, BASE_CHUNKS)

    def fire_idx(k, p):
        row0 = (wid + k * NW) * NSUB
        pltpu.async_copy(src2_h.at[pl.ds(row0, NSUB)], src_v.at[p], sem_idx)
        pltpu.async_copy(dst2_h.at[pl.ds(row0, NSUB)], dst_v.at[p], sem_idx)

    def drain_idx(p):
        pltpu.make_async_copy(src2_h.at[pl.ds(0, NSUB)], src_v.at[p],
                              sem_idx).wait()
        pltpu.make_async_copy(dst2_h.at[pl.ds(0, NSUB)], dst_v.at[p],
                              sem_idx).wait()

    fire_idx(0, 0)

    def chunk_body(k, carry):
        p = lax.rem(k, 2)
        chunk = wid + k * NW
        ebase = chunk * CHUNK
        drain_idx(p)

        @pl.when(k + 1 < nmine)
        def _():
            fire_idx(k + 1, 1 - p)

        copies = []
        for j in range(NSUB):
            cu = pltpu.async_copy(eu_h.at[src_v.at[p, j]], u_v.at[j],
                                  gsems[j])
            cv = pltpu.async_copy(ev_h.at[dst_v.at[p, j]], v_v.at[j],
                                  gsems[j])
            copies.append((cu, cv))

        outs = []
        for j in range(NSUB):
            cu, cv = copies[j]
            cu.wait()
            cv.wait()

            @plsc.parallel_loop(0, SUB // L, unroll=4)
            def grp(kg, j=j):
                eidx = kg * L + lax.iota(jnp.int32, L)
                acc = jnp.zeros((L,), jnp.float32)
                for d in range(D):
                    dsp = jnp.full((L,), d, jnp.int32)
                    ud = plsc.load_gather(u_v.at[j], [eidx, dsp])
                    vd = plsc.load_gather(v_v.at[j], [eidx, dsp])
                    acc = acc + ud * vd * g2_v[d]
                t = jnp.exp(acc)
                ag = 1.0 / (1.0 + jnp.exp(-t))
                ag_v[j, pl.ds(kg * L, L)] = ag
            outs.append(pltpu.async_copy(
                ag_v.at[j], sums_s.at[src_v.at[p, j]], sem_a, add=True))
            outs.append(pltpu.async_copy(
                ag_v.at[j], ag_h.at[pl.ds(ebase + j * SUB, SUB)], sem_s))
        for c in outs:
            c.wait()
        return carry

    lax.fori_loop(0, nmine, chunk_body, 0)
    plsc.subcore_barrier()

    for c in range(NC):
        psum_h = (psum0_h, psum1_h)[c]

        @pl.when(jnp.logical_and(cid == c, sid < NS - 1))
        def _():
            pltpu.sync_copy(sums_s.at[pl.ds(sid * DUMP, DUMP)],
                            psum_h.at[pl.ds(sid * DUMP, DUMP)])

        @pl.when(jnp.logical_and(cid == c, sid == NS - 1))
        def _():
            pltpu.sync_copy(sums_s.at[pl.ds((NS - 1) * DUMP, DUMP_LAST)],
                            psum_h.at[pl.ds((NS - 1) * DUMP, DUMP_LAST)])


_edge_kernel = functools.partial(
    pl.kernel,
    out_type=(
        jax.ShapeDtypeStruct((N_EDGES,), jnp.float32),
        jax.ShapeDtypeStruct((N_NODES,), jnp.float32),
        jax.ShapeDtypeStruct((N_NODES,), jnp.float32),
    ),
    mesh=_mesh,
    scratch_types=[
        pltpu.VMEM((2, NSUB, SUB), jnp.int32),
        pltpu.VMEM((2, NSUB, SUB), jnp.int32),
        pltpu.VMEM((NSUB, SUB, D), jnp.float32),
        pltpu.VMEM((NSUB, SUB, D), jnp.float32),
        pltpu.VMEM((NSUB, SUB), jnp.float32),
        pltpu.VMEM((D, L), jnp.float32),
        pltpu.VMEM_SHARED((N_NODES,), jnp.float32),
        pltpu.SemaphoreType.DMA,
        pltpu.SemaphoreType.DMA,
        pltpu.SemaphoreType.DMA,
        pltpu.SemaphoreType.DMA,
        pltpu.SemaphoreType.DMA,
        pltpu.SemaphoreType.DMA,
        pltpu.SemaphoreType.DMA,
    ],
    compiler_params=pltpu.CompilerParams(needs_layout_passes=False,
                                         use_tc_tiling_on_sc=False),
)(_edge_body)


def _norm_body(src2_h, ag_h, p0_h, p1_h,
               c_h,
               src_v, s0_v, s1_v, ag_v, c_v,
               sem_idx, sem_g0, sem_g1, sem_g2, sem_g3, sem_s):
    cid = lax.axis_index("c")
    sid = lax.axis_index("s")
    wid = sid * NC + cid
    gsems = (sem_g0, sem_g1, sem_g2, sem_g3)
    nmine = jnp.where(wid < EXTRA, BASE_CHUNKS + 1, BASE_CHUNKS)

    def fire_idx(k, p):
        row0 = (wid + k * NW) * NSUB
        pltpu.async_copy(src2_h.at[pl.ds(row0, NSUB)], src_v.at[p], sem_idx)

    def drain_idx(p):
        pltpu.make_async_copy(src2_h.at[pl.ds(0, NSUB)], src_v.at[p],
                              sem_idx).wait()

    fire_idx(0, 0)

    def chunk_body(k, carry):
        p = lax.rem(k, 2)
        chunk = wid + k * NW
        ebase = chunk * CHUNK
        drain_idx(p)

        @pl.when(k + 1 < nmine)
        def _():
            fire_idx(k + 1, 1 - p)

        copies = []
        for j in range(NSUB):
            c0 = pltpu.async_copy(p0_h.at[src_v.at[p, j]], s0_v.at[j],
                                  gsems[j])
            c1 = pltpu.async_copy(p1_h.at[src_v.at[p, j]], s1_v.at[j],
                                  gsems[j])
            ca = pltpu.async_copy(ag_h.at[pl.ds(ebase + j * SUB, SUB)],
                                  ag_v.at[j], gsems[j])
            copies.append((c0, c1, ca))

        outs = []
        for j in range(NSUB):
            for c in copies[j]:
                c.wait()

            def grp(kg, c2, j=j):
                sl = pl.ds(kg * L, L)
                ag = ag_v[j, sl]
                s = s0_v[j, sl] + s1_v[j, sl]
                c_v[j, sl] = jnp.minimum(jnp.maximum(ag * 5.0 / s, 0.0), 1.0)
                return c2

            lax.fori_loop(0, SUB // L, grp, 0)
            outs.append(pltpu.async_copy(
                c_v.at[j], c_h.at[pl.ds(ebase + j * SUB, SUB)], sem_s))
        for c in outs:
            c.wait()
        return carry

    lax.fori_loop(0, nmine, chunk_body, 0)


_norm_kernel = functools.partial(
    pl.kernel,
    out_type=jax.ShapeDtypeStruct((N_EDGES,), jnp.float32),
    mesh=_mesh,
    scratch_types=[
        pltpu.VMEM((2, NSUB, SUB), jnp.int32),
        pltpu.VMEM((NSUB, SUB), jnp.float32),
        pltpu.VMEM((NSUB, SUB), jnp.float32),
        pltpu.VMEM((NSUB, SUB), jnp.float32),
        pltpu.VMEM((NSUB, SUB), jnp.float32),
        pltpu.SemaphoreType.DMA,
        pltpu.SemaphoreType.DMA,
        pltpu.SemaphoreType.DMA,
        pltpu.SemaphoreType.DMA,
        pltpu.SemaphoreType.DMA,
        pltpu.SemaphoreType.DMA,
    ],
    compiler_params=pltpu.CompilerParams(needs_layout_passes=False,
                                         use_tc_tiling_on_sc=False),
)(_norm_body)


def kernel(Eu, Ev, edge_index, g):
    src = edge_index[0]
    dst = edge_index[1]
    src2 = src.reshape(N_EDGES // SUB, SUB)
    dst2 = dst.reshape(N_EDGES // SUB, SUB)
    g2 = (g * g).reshape(D)
    g2b = jnp.broadcast_to(g2[:, None], (D, L))
    zeros = jnp.zeros((N_NODES,), jnp.float32)
    ag, psum0, psum1 = _edge_kernel(src2, dst2, Eu, Ev, g2b, zeros)
    return _norm_kernel(src2, ag, psum0, psum1)


# speedup vs baseline: 2.2885x; 2.2885x over previous
import functools

import jax
import jax.numpy as jnp
from jax import lax
from jax.experimental import pallas as pl
from jax.experimental.pallas import tpu as pltpu
from jax.experimental.pallas import tpu_sc as plsc

N_NODES = 100000
N_EDGES = 1600000
D = 32
L = 16
NC = 2
NS = 16
NW = NC * NS
SUB = 128
NSUB = 4
CHUNK = SUB * NSUB
NCHUNKS = N_EDGES // CHUNK
BASE_CHUNKS = NCHUNKS // NW
EXTRA = NCHUNKS - BASE_CHUNKS * NW
DUMP = 6256
DUMP_LAST = N_NODES - 15 * DUMP

_mesh = plsc.VectorSubcoreMesh(core_axis_name="c", subcore_axis_name="s")


def _edge_body(src2_h, dst2_h, eu_h, ev_h, g2b_h, zeros_h,
               ag_h, psum0_h, psum1_h,
               src_v, dst_v, u_v, v_v, ag_v, g2_v, sums_s,
               sem_idx, sem_g0, sem_g1, sem_g2, sem_g3, sem_s, sem_a):
    cid = lax.axis_index("c")
    sid = lax.axis_index("s")
    wid = sid * NC + cid
    gsems = (sem_g0, sem_g1, sem_g2, sem_g3)

    @pl.when(sid == 0)
    def _():
        pltpu.sync_copy(zeros_h, sums_s)

    pltpu.sync_copy(g2b_h, g2_v)
    plsc.subcore_barrier()

    nmine = jnp.where(wid < EXTRA, BASE_CHUNKS + 1, BASE_CHUNKS)

    def fire_idx(k, p):
        row0 = (wid + k * NW) * NSUB
        pltpu.async_copy(src2_h.at[pl.ds(row0, NSUB)], src_v.at[p], sem_idx)
        pltpu.async_copy(dst2_h.at[pl.ds(row0, NSUB)], dst_v.at[p], sem_idx)

    def drain_idx(p):
        pltpu.make_async_copy(src2_h.at[pl.ds(0, NSUB)], src_v.at[p],
                              sem_idx).wait()
        pltpu.make_async_copy(dst2_h.at[pl.ds(0, NSUB)], dst_v.at[p],
                              sem_idx).wait()

    fire_idx(0, 0)

    def chunk_body(k, carry):
        p = lax.rem(k, 2)
        chunk = wid + k * NW
        ebase = chunk * CHUNK
        drain_idx(p)

        @pl.when(k + 1 < nmine)
        def _():
            fire_idx(k + 1, 1 - p)

        copies = []
        for j in range(NSUB):
            cu = pltpu.async_copy(eu_h.at[src_v.at[p, j]], u_v.at[j],
                                  gsems[j])
            cv = pltpu.async_copy(ev_h.at[dst_v.at[p, j]], v_v.at[j],
                                  gsems[j])
            copies.append((cu, cv))

        outs = []
        for j in range(NSUB):
            cu, cv = copies[j]
            cu.wait()
            cv.wait()

            @plsc.parallel_loop(0, SUB // L, unroll=4)
            def grp(kg, j=j):
                lane = lax.iota(jnp.int32, L)
                eidx = kg * L + lane
                acc = jnp.zeros((L,), jnp.float32)
                for d in range(D):
                    dsp = (d + lane) & (D - 1)
                    ud = plsc.load_gather(u_v.at[j], [eidx, dsp])
                    vd = plsc.load_gather(v_v.at[j], [eidx, dsp])
                    acc = acc + ud * vd * g2_v[d]
                t = jnp.exp(acc)
                ag = 1.0 / (1.0 + jnp.exp(-t))
                ag_v[j, pl.ds(kg * L, L)] = ag
            outs.append(pltpu.async_copy(
                ag_v.at[j], sums_s.at[src_v.at[p, j]], sem_a, add=True))
            outs.append(pltpu.async_copy(
                ag_v.at[j], ag_h.at[pl.ds(ebase + j * SUB, SUB)], sem_s))
        for c in outs:
            c.wait()
        return carry

    lax.fori_loop(0, nmine, chunk_body, 0)
    plsc.subcore_barrier()

    for c in range(NC):
        psum_h = (psum0_h, psum1_h)[c]

        @pl.when(jnp.logical_and(cid == c, sid < NS - 1))
        def _():
            pltpu.sync_copy(sums_s.at[pl.ds(sid * DUMP, DUMP)],
                            psum_h.at[pl.ds(sid * DUMP, DUMP)])

        @pl.when(jnp.logical_and(cid == c, sid == NS - 1))
        def _():
            pltpu.sync_copy(sums_s.at[pl.ds((NS - 1) * DUMP, DUMP_LAST)],
                            psum_h.at[pl.ds((NS - 1) * DUMP, DUMP_LAST)])


_edge_kernel = functools.partial(
    pl.kernel,
    out_type=(
        jax.ShapeDtypeStruct((N_EDGES,), jnp.float32),
        jax.ShapeDtypeStruct((N_NODES,), jnp.float32),
        jax.ShapeDtypeStruct((N_NODES,), jnp.float32),
    ),
    mesh=_mesh,
    scratch_types=[
        pltpu.VMEM((2, NSUB, SUB), jnp.int32),
        pltpu.VMEM((2, NSUB, SUB), jnp.int32),
        pltpu.VMEM((NSUB, SUB, D), jnp.float32),
        pltpu.VMEM((NSUB, SUB, D), jnp.float32),
        pltpu.VMEM((NSUB, SUB), jnp.float32),
        pltpu.VMEM((D, L), jnp.float32),
        pltpu.VMEM_SHARED((N_NODES,), jnp.float32),
        pltpu.SemaphoreType.DMA,
        pltpu.SemaphoreType.DMA,
        pltpu.SemaphoreType.DMA,
        pltpu.SemaphoreType.DMA,
        pltpu.SemaphoreType.DMA,
        pltpu.SemaphoreType.DMA,
        pltpu.SemaphoreType.DMA,
    ],
    compiler_params=pltpu.CompilerParams(needs_layout_passes=False,
                                         use_tc_tiling_on_sc=False),
)(_edge_body)


def _norm_body(src2_h, ag_h, p0_h, p1_h,
               c_h,
               src_v, s0_v, s1_v, ag_v, c_v,
               sem_idx, sem_g0, sem_g1, sem_g2, sem_g3, sem_s):
    cid = lax.axis_index("c")
    sid = lax.axis_index("s")
    wid = sid * NC + cid
    gsems = (sem_g0, sem_g1, sem_g2, sem_g3)
    nmine = jnp.where(wid < EXTRA, BASE_CHUNKS + 1, BASE_CHUNKS)

    def fire_idx(k, p):
        row0 = (wid + k * NW) * NSUB
        pltpu.async_copy(src2_h.at[pl.ds(row0, NSUB)], src_v.at[p], sem_idx)

    def drain_idx(p):
        pltpu.make_async_copy(src2_h.at[pl.ds(0, NSUB)], src_v.at[p],
                              sem_idx).wait()

    fire_idx(0, 0)

    def chunk_body(k, carry):
        p = lax.rem(k, 2)
        chunk = wid + k * NW
        ebase = chunk * CHUNK
        drain_idx(p)

        @pl.when(k + 1 < nmine)
        def _():
            fire_idx(k + 1, 1 - p)

        copies = []
        for j in range(NSUB):
            c0 = pltpu.async_copy(p0_h.at[src_v.at[p, j]], s0_v.at[j],
                                  gsems[j])
            c1 = pltpu.async_copy(p1_h.at[src_v.at[p, j]], s1_v.at[j],
                                  gsems[j])
            ca = pltpu.async_copy(ag_h.at[pl.ds(ebase + j * SUB, SUB)],
                                  ag_v.at[j], gsems[j])
            copies.append((c0, c1, ca))

        outs = []
        for j in range(NSUB):
            for c in copies[j]:
                c.wait()

            def grp(kg, c2, j=j):
                sl = pl.ds(kg * L, L)
                ag = ag_v[j, sl]
                s = s0_v[j, sl] + s1_v[j, sl]
                c_v[j, sl] = jnp.minimum(jnp.maximum(ag * 5.0 / s, 0.0), 1.0)
                return c2

            lax.fori_loop(0, SUB // L, grp, 0)
            outs.append(pltpu.async_copy(
                c_v.at[j], c_h.at[pl.ds(ebase + j * SUB, SUB)], sem_s))
        for c in outs:
            c.wait()
        return carry

    lax.fori_loop(0, nmine, chunk_body, 0)


_norm_kernel = functools.partial(
    pl.kernel,
    out_type=jax.ShapeDtypeStruct((N_EDGES,), jnp.float32),
    mesh=_mesh,
    scratch_types=[
        pltpu.VMEM((2, NSUB, SUB), jnp.int32),
        pltpu.VMEM((NSUB, SUB), jnp.float32),
        pltpu.VMEM((NSUB, SUB), jnp.float32),
        pltpu.VMEM((NSUB, SUB), jnp.float32),
        pltpu.VMEM((NSUB, SUB), jnp.float32),
        pltpu.SemaphoreType.DMA,
        pltpu.SemaphoreType.DMA,
        pltpu.SemaphoreType.DMA,
        pltpu.SemaphoreType.DMA,
        pltpu.SemaphoreType.DMA,
        pltpu.SemaphoreType.DMA,
    ],
    compiler_params=pltpu.CompilerParams(needs_layout_passes=False,
                                         use_tc_tiling_on_sc=False),
)(_norm_body)


def kernel(Eu, Ev, edge_index, g):
    src = edge_index[0]
    dst = edge_index[1]
    src2 = src.reshape(N_EDGES // SUB, SUB)
    dst2 = dst.reshape(N_EDGES // SUB, SUB)
    g2 = (g * g).reshape(D)
    dl = (jnp.arange(D)[:, None] + jnp.arange(L)[None, :]) % D
    g2b = g2[dl]
    zeros = jnp.zeros((N_NODES,), jnp.float32)
    ag, psum0, psum1 = _edge_kernel(src2, dst2, Eu, Ev, g2b, zeros)
    return _norm_kernel(src2, ag, psum0, psum1)


# speedup vs baseline: 2.7097x; 1.1840x over previous
import functools

import jax
import jax.numpy as jnp
from jax import lax
from jax.experimental import pallas as pl
from jax.experimental.pallas import tpu as pltpu
from jax.experimental.pallas import tpu_sc as plsc

N_NODES = 100000
N_EDGES = 1600000
D = 32
L = 16
NC = 2
NS = 16
NW = NC * NS
SUB = 128
NSUB = 4
CHUNK = SUB * NSUB
NCHUNKS = N_EDGES // CHUNK
BASE_CHUNKS = NCHUNKS // NW
EXTRA = NCHUNKS - BASE_CHUNKS * NW
DUMP = 6256
DUMP_LAST = N_NODES - 15 * DUMP

_mesh = plsc.VectorSubcoreMesh(core_axis_name="c", subcore_axis_name="s")


def _edge_body(src2_h, dst2_h, eu_h, ev_h, g2b_h, zeros_h,
               ag_h, psum0_h, psum1_h,
               src_v, dst_v, u_v, v_v, ag_v, g2_v, sums_s,
               sem_idx, sem_ga0, sem_ga1, sem_ga2, sem_ga3,
               sem_gb0, sem_gb1, sem_gb2, sem_gb3, sem_s, sem_a):
    cid = lax.axis_index("c")
    sid = lax.axis_index("s")
    wid = sid * NC + cid
    gsems = ((sem_ga0, sem_ga1, sem_ga2, sem_ga3),
             (sem_gb0, sem_gb1, sem_gb2, sem_gb3))

    @pl.when(sid == 0)
    def _():
        pltpu.sync_copy(zeros_h, sums_s)

    pltpu.sync_copy(g2b_h, g2_v)
    plsc.subcore_barrier()

    nmine = jnp.where(wid < EXTRA, BASE_CHUNKS + 1, BASE_CHUNKS)

    def fire_idx(k):
        r = lax.rem(k, 4)
        row0 = (wid + k * NW) * NSUB
        pltpu.async_copy(src2_h.at[pl.ds(row0, NSUB)], src_v.at[r], sem_idx)
        pltpu.async_copy(dst2_h.at[pl.ds(row0, NSUB)], dst_v.at[r], sem_idx)

    def drain_idx(k):
        r = lax.rem(k, 4)
        pltpu.make_async_copy(src2_h.at[pl.ds(0, NSUB)], src_v.at[r],
                              sem_idx).wait()
        pltpu.make_async_copy(dst2_h.at[pl.ds(0, NSUB)], dst_v.at[r],
                              sem_idx).wait()

    def fire_gathers(k):
        r = lax.rem(k, 4)
        p = lax.rem(k, 2)
        for j in range(NSUB):
            for pp in range(2):
                @pl.when(p == pp)
                def _(pp=pp, j=j):
                    pltpu.async_copy(eu_h.at[src_v.at[r, j]], u_v.at[pp, j],
                                     gsems[pp][j])
                    pltpu.async_copy(ev_h.at[dst_v.at[r, j]], v_v.at[pp, j],
                                     gsems[pp][j])

    def drain_outs(k):
        r = lax.rem(k, 4)
        p = lax.rem(k, 2)
        chunk = wid + k * NW
        ebase = chunk * CHUNK
        for j in range(NSUB):
            pltpu.make_async_copy(
                ag_v.at[p, j], sums_s.at[src_v.at[r, j]], sem_a).wait()
            pltpu.make_async_copy(
                ag_v.at[p, j], ag_h.at[pl.ds(ebase + j * SUB, SUB)],
                sem_s).wait()

    fire_idx(0)
    drain_idx(0)

    @pl.when(1 < nmine)
    def _():
        fire_idx(1)
    fire_gathers(0)

    def chunk_body(k, carry):
        r = lax.rem(k, 4)
        p = lax.rem(k, 2)
        chunk = wid + k * NW
        ebase = chunk * CHUNK

        @pl.when(k >= 1)
        def _():
            drain_outs(k - 1)

        @pl.when(k + 1 < nmine)
        def _():
            drain_idx(k + 1)

            @pl.when(k + 2 < nmine)
            def _():
                fire_idx(k + 2)
            fire_gathers(k + 1)

        for j in range(NSUB):
            for pp in range(2):
                @pl.when(p == pp)
                def _(pp=pp, j=j):
                    pltpu.make_async_copy(eu_h.at[src_v.at[r, j]],
                                          u_v.at[pp, j], gsems[pp][j]).wait()
                    pltpu.make_async_copy(ev_h.at[dst_v.at[r, j]],
                                          v_v.at[pp, j], gsems[pp][j]).wait()

            @plsc.parallel_loop(0, SUB // L, unroll=4)
            def grp(kg, j=j):
                lane = lax.iota(jnp.int32, L)
                eidx = kg * L + lane
                acc = jnp.zeros((L,), jnp.float32)
                for d in range(D):
                    dsp = (d + lane) & (D - 1)
                    ud = plsc.load_gather(u_v.at[p, j], [eidx, dsp])
                    vd = plsc.load_gather(v_v.at[p, j], [eidx, dsp])
                    acc = acc + ud * vd * g2_v[d]
                t = jnp.exp(acc)
                ag = 1.0 / (1.0 + jnp.exp(-t))
                ag_v[p, j, pl.ds(kg * L, L)] = ag

            pltpu.async_copy(ag_v.at[p, j], sums_s.at[src_v.at[r, j]],
                             sem_a, add=True)
            pltpu.async_copy(ag_v.at[p, j],
                             ag_h.at[pl.ds(ebase + j * SUB, SUB)], sem_s)
        return carry

    lax.fori_loop(0, nmine, chunk_body, 0)
    drain_outs(nmine - 1)
    plsc.subcore_barrier()

    for c in range(NC):
        psum_h = (psum0_h, psum1_h)[c]

        @pl.when(jnp.logical_and(cid == c, sid < NS - 1))
        def _():
            pltpu.sync_copy(sums_s.at[pl.ds(sid * DUMP, DUMP)],
                            psum_h.at[pl.ds(sid * DUMP, DUMP)])

        @pl.when(jnp.logical_and(cid == c, sid == NS - 1))
        def _():
            pltpu.sync_copy(sums_s.at[pl.ds((NS - 1) * DUMP, DUMP_LAST)],
                            psum_h.at[pl.ds((NS - 1) * DUMP, DUMP_LAST)])


_edge_kernel = functools.partial(
    pl.kernel,
    out_type=(
        jax.ShapeDtypeStruct((N_EDGES,), jnp.float32),
        jax.ShapeDtypeStruct((N_NODES,), jnp.float32),
        jax.ShapeDtypeStruct((N_NODES,), jnp.float32),
    ),
    mesh=_mesh,
    scratch_types=[
        pltpu.VMEM((4, NSUB, SUB), jnp.int32),
        pltpu.VMEM((4, NSUB, SUB), jnp.int32),
        pltpu.VMEM((2, NSUB, SUB, D), jnp.float32),
        pltpu.VMEM((2, NSUB, SUB, D), jnp.float32),
        pltpu.VMEM((2, NSUB, SUB), jnp.float32),
        pltpu.VMEM((D, L), jnp.float32),
        pltpu.VMEM_SHARED((N_NODES,), jnp.float32),
    ] + [pltpu.SemaphoreType.DMA] * 11,
    compiler_params=pltpu.CompilerParams(needs_layout_passes=False,
                                         use_tc_tiling_on_sc=False),
)(_edge_body)


def _norm_body(src2_h, ag_h, p0_h, p1_h,
               c_h,
               src_v, s0_v, s1_v, ag_v, c_v,
               sem_idx, sem_g0, sem_g1, sem_g2, sem_g3, sem_s):
    cid = lax.axis_index("c")
    sid = lax.axis_index("s")
    wid = sid * NC + cid
    gsems = (sem_g0, sem_g1, sem_g2, sem_g3)
    nmine = jnp.where(wid < EXTRA, BASE_CHUNKS + 1, BASE_CHUNKS)

    def fire_idx(k, p):
        row0 = (wid + k * NW) * NSUB
        pltpu.async_copy(src2_h.at[pl.ds(row0, NSUB)], src_v.at[p], sem_idx)

    def drain_idx(p):
        pltpu.make_async_copy(src2_h.at[pl.ds(0, NSUB)], src_v.at[p],
                              sem_idx).wait()

    fire_idx(0, 0)

    def chunk_body(k, carry):
        p = lax.rem(k, 2)
        chunk = wid + k * NW
        ebase = chunk * CHUNK
        drain_idx(p)

        @pl.when(k + 1 < nmine)
        def _():
            fire_idx(k + 1, 1 - p)

        copies = []
        for j in range(NSUB):
            c0 = pltpu.async_copy(p0_h.at[src_v.at[p, j]], s0_v.at[j],
                                  gsems[j])
            c1 = pltpu.async_copy(p1_h.at[src_v.at[p, j]], s1_v.at[j],
                                  gsems[j])
            ca = pltpu.async_copy(ag_h.at[pl.ds(ebase + j * SUB, SUB)],
                                  ag_v.at[j], gsems[j])
            copies.append((c0, c1, ca))

        outs = []
        for j in range(NSUB):
            for c in copies[j]:
                c.wait()

            def grp(kg, c2, j=j):
                sl = pl.ds(kg * L, L)
                ag = ag_v[j, sl]
                s = s0_v[j, sl] + s1_v[j, sl]
                c_v[j, sl] = jnp.minimum(jnp.maximum(ag * 5.0 / s, 0.0), 1.0)
                return c2

            lax.fori_loop(0, SUB // L, grp, 0)
            outs.append(pltpu.async_copy(
                c_v.at[j], c_h.at[pl.ds(ebase + j * SUB, SUB)], sem_s))
        for c in outs:
            c.wait()
        return carry

    lax.fori_loop(0, nmine, chunk_body, 0)


_norm_kernel = functools.partial(
    pl.kernel,
    out_type=jax.ShapeDtypeStruct((N_EDGES,), jnp.float32),
    mesh=_mesh,
    scratch_types=[
        pltpu.VMEM((2, NSUB, SUB), jnp.int32),
        pltpu.VMEM((NSUB, SUB), jnp.float32),
        pltpu.VMEM((NSUB, SUB), jnp.float32),
        pltpu.VMEM((NSUB, SUB), jnp.float32),
        pltpu.VMEM((NSUB, SUB), jnp.float32),
        pltpu.SemaphoreType.DMA,
        pltpu.SemaphoreType.DMA,
        pltpu.SemaphoreType.DMA,
        pltpu.SemaphoreType.DMA,
        pltpu.SemaphoreType.DMA,
        pltpu.SemaphoreType.DMA,
    ],
    compiler_params=pltpu.CompilerParams(needs_layout_passes=False,
                                         use_tc_tiling_on_sc=False),
)(_norm_body)


def kernel(Eu, Ev, edge_index, g):
    src = edge_index[0]
    dst = edge_index[1]
    src2 = src.reshape(N_EDGES // SUB, SUB)
    dst2 = dst.reshape(N_EDGES // SUB, SUB)
    g2 = (g * g).reshape(D)
    dl = (jnp.arange(D)[:, None] + jnp.arange(L)[None, :]) % D
    g2b = g2[dl]
    zeros = jnp.zeros((N_NODES,), jnp.float32)
    ag, psum0, psum1 = _edge_kernel(src2, dst2, Eu, Ev, g2b, zeros)
    return _norm_kernel(src2, ag, psum0, psum1)


# speedup vs baseline: 2.8122x; 1.0379x over previous
import functools

import jax
import jax.numpy as jnp
from jax import lax
from jax.experimental import pallas as pl
from jax.experimental.pallas import tpu as pltpu
from jax.experimental.pallas import tpu_sc as plsc

N_NODES = 100000
N_EDGES = 1600000
D = 32
L = 16
NC = 2
NS = 16
NW = NC * NS
SUB = 128
NSUB = 4
CHUNK = SUB * NSUB
NCHUNKS = N_EDGES // CHUNK
BASE_CHUNKS = NCHUNKS // NW
EXTRA = NCHUNKS - BASE_CHUNKS * NW
DUMP = 6256
DUMP_LAST = N_NODES - 15 * DUMP
NSUB2 = 20
CHUNK2 = SUB * NSUB2
NCHUNKS2 = N_EDGES // CHUNK2
BASE2 = NCHUNKS2 // NW
EXTRA2 = NCHUNKS2 - BASE2 * NW

_mesh = plsc.VectorSubcoreMesh(core_axis_name="c", subcore_axis_name="s")


def _edge_body(src2_h, dst2_h, eu_h, ev_h, g2b_h, zeros_h,
               ag_h, psum0_h, psum1_h,
               src_v, dst_v, u_v, v_v, ag_v, g2_v, sums_s,
               sem_idx, sem_ga0, sem_ga1, sem_ga2, sem_ga3,
               sem_gb0, sem_gb1, sem_gb2, sem_gb3, sem_s, sem_a):
    cid = lax.axis_index("c")
    sid = lax.axis_index("s")
    wid = sid * NC + cid
    gsems = ((sem_ga0, sem_ga1, sem_ga2, sem_ga3),
             (sem_gb0, sem_gb1, sem_gb2, sem_gb3))

    @pl.when(sid == 0)
    def _():
        pltpu.sync_copy(zeros_h, sums_s)

    pltpu.sync_copy(g2b_h, g2_v)
    plsc.subcore_barrier()

    nmine = jnp.where(wid < EXTRA, BASE_CHUNKS + 1, BASE_CHUNKS)

    def fire_idx(k):
        r = lax.rem(k, 4)
        row0 = (wid + k * NW) * NSUB
        pltpu.async_copy(src2_h.at[pl.ds(row0, NSUB)], src_v.at[r], sem_idx)
        pltpu.async_copy(dst2_h.at[pl.ds(row0, NSUB)], dst_v.at[r], sem_idx)

    def drain_idx(k):
        r = lax.rem(k, 4)
        pltpu.make_async_copy(src2_h.at[pl.ds(0, NSUB)], src_v.at[r],
                              sem_idx).wait()
        pltpu.make_async_copy(dst2_h.at[pl.ds(0, NSUB)], dst_v.at[r],
                              sem_idx).wait()

    def fire_gathers(k):
        r = lax.rem(k, 4)
        p = lax.rem(k, 2)
        for j in range(NSUB):
            for pp in range(2):
                @pl.when(p == pp)
                def _(pp=pp, j=j):
                    pltpu.async_copy(eu_h.at[src_v.at[r, j]], u_v.at[pp, j],
                                     gsems[pp][j])
                    pltpu.async_copy(ev_h.at[dst_v.at[r, j]], v_v.at[pp, j],
                                     gsems[pp][j])

    def drain_outs(k):
        r = lax.rem(k, 4)
        p = lax.rem(k, 2)
        chunk = wid + k * NW
        ebase = chunk * CHUNK
        for j in range(NSUB):
            pltpu.make_async_copy(
                ag_v.at[p, j], sums_s.at[src_v.at[r, j]], sem_a).wait()
            pltpu.make_async_copy(
                ag_v.at[p, j], ag_h.at[pl.ds(ebase + j * SUB, SUB)],
                sem_s).wait()

    fire_idx(0)
    drain_idx(0)

    @pl.when(1 < nmine)
    def _():
        fire_idx(1)
    fire_gathers(0)

    def chunk_body(k, carry):
        r = lax.rem(k, 4)
        p = lax.rem(k, 2)
        chunk = wid + k * NW
        ebase = chunk * CHUNK

        @pl.when(k >= 1)
        def _():
            drain_outs(k - 1)

        @pl.when(k + 1 < nmine)
        def _():
            drain_idx(k + 1)

            @pl.when(k + 2 < nmine)
            def _():
                fire_idx(k + 2)
            fire_gathers(k + 1)

        for j in range(NSUB):
            for pp in range(2):
                @pl.when(p == pp)
                def _(pp=pp, j=j):
                    pltpu.make_async_copy(eu_h.at[src_v.at[r, j]],
                                          u_v.at[pp, j], gsems[pp][j]).wait()
                    pltpu.make_async_copy(ev_h.at[dst_v.at[r, j]],
                                          v_v.at[pp, j], gsems[pp][j]).wait()

            @plsc.parallel_loop(0, SUB // L, unroll=4)
            def grp(kg, j=j):
                lane = lax.iota(jnp.int32, L)
                eidx = kg * L + lane
                acc = jnp.zeros((L,), jnp.float32)
                for d in range(D):
                    dsp = (d + lane) & (D - 1)
                    ud = plsc.load_gather(u_v.at[p, j], [eidx, dsp])
                    vd = plsc.load_gather(v_v.at[p, j], [eidx, dsp])
                    acc = acc + ud * vd * g2_v[d]
                t = jnp.exp(acc)
                ag = 1.0 / (1.0 + jnp.exp(-t))
                ag_v[p, j, pl.ds(kg * L, L)] = ag

            pltpu.async_copy(ag_v.at[p, j], sums_s.at[src_v.at[r, j]],
                             sem_a, add=True)
            pltpu.async_copy(ag_v.at[p, j],
                             ag_h.at[pl.ds(ebase + j * SUB, SUB)], sem_s)
        return carry

    lax.fori_loop(0, nmine, chunk_body, 0)
    drain_outs(nmine - 1)
    plsc.subcore_barrier()

    for c in range(NC):
        psum_h = (psum0_h, psum1_h)[c]

        @pl.when(jnp.logical_and(cid == c, sid < NS - 1))
        def _():
            pltpu.sync_copy(sums_s.at[pl.ds(sid * DUMP, DUMP)],
                            psum_h.at[pl.ds(sid * DUMP, DUMP)])

        @pl.when(jnp.logical_and(cid == c, sid == NS - 1))
        def _():
            pltpu.sync_copy(sums_s.at[pl.ds((NS - 1) * DUMP, DUMP_LAST)],
                            psum_h.at[pl.ds((NS - 1) * DUMP, DUMP_LAST)])


_edge_kernel = functools.partial(
    pl.kernel,
    out_type=(
        jax.ShapeDtypeStruct((N_EDGES,), jnp.float32),
        jax.ShapeDtypeStruct((N_NODES,), jnp.float32),
        jax.ShapeDtypeStruct((N_NODES,), jnp.float32),
    ),
    mesh=_mesh,
    scratch_types=[
        pltpu.VMEM((4, NSUB, SUB), jnp.int32),
        pltpu.VMEM((4, NSUB, SUB), jnp.int32),
        pltpu.VMEM((2, NSUB, SUB, D), jnp.float32),
        pltpu.VMEM((2, NSUB, SUB, D), jnp.float32),
        pltpu.VMEM((2, NSUB, SUB), jnp.float32),
        pltpu.VMEM((D, L), jnp.float32),
        pltpu.VMEM_SHARED((N_NODES,), jnp.float32),
    ] + [pltpu.SemaphoreType.DMA] * 11,
    compiler_params=pltpu.CompilerParams(needs_layout_passes=False,
                                         use_tc_tiling_on_sc=False),
)(_edge_body)


def _norm_body(src2_h, ag_h, p0_h, p1_h,
               c_h,
               src_v, s0_v, s1_v, ag_v, c_v,
               sem_idx, sem_ga, sem_gb, sem_s):
    cid = lax.axis_index("c")
    sid = lax.axis_index("s")
    wid = sid * NC + cid
    gsems = (sem_ga, sem_gb)
    nmine = jnp.where(wid < EXTRA2, BASE2 + 1, BASE2)

    def fire_idx(k):
        r = lax.rem(k, 4)
        row0 = (wid + k * NW) * NSUB2
        pltpu.async_copy(src2_h.at[pl.ds(row0, NSUB2)], src_v.at[r], sem_idx)

    def drain_idx(k):
        r = lax.rem(k, 4)
        pltpu.make_async_copy(src2_h.at[pl.ds(0, NSUB2)], src_v.at[r],
                              sem_idx).wait()

    def transfers(k, p):
        r = lax.rem(k, 4)
        ebase = (wid + k * NW) * CHUNK2
        out = []
        for j in range(NSUB2):
            out.append(pltpu.make_async_copy(
                p0_h.at[src_v.at[r, j]], s0_v.at[p, j], gsems[p]))
            out.append(pltpu.make_async_copy(
                p1_h.at[src_v.at[r, j]], s1_v.at[p, j], gsems[p]))
            out.append(pltpu.make_async_copy(
                ag_h.at[pl.ds(ebase + j * SUB, SUB)], ag_v.at[p, j],
                gsems[p]))
        return out

    def fire_gathers(k):
        for pp in range(2):
            @pl.when(lax.rem(k, 2) == pp)
            def _(pp=pp):
                for c in transfers(k, pp):
                    c.start()

    def drain_gathers(k):
        for pp in range(2):
            @pl.when(lax.rem(k, 2) == pp)
            def _(pp=pp):
                for c in transfers(k, pp):
                    c.wait()

    def drain_stores(k):
        p = lax.rem(k, 2)
        ebase = (wid + k * NW) * CHUNK2
        for j in range(NSUB2):
            pltpu.make_async_copy(
                c_v.at[p, j], c_h.at[pl.ds(ebase + j * SUB, SUB)],
                sem_s).wait()

    fire_idx(0)
    drain_idx(0)

    @pl.when(1 < nmine)
    def _():
        fire_idx(1)
    fire_gathers(0)

    def chunk_body(k, carry):
        p = lax.rem(k, 2)
        ebase = (wid + k * NW) * CHUNK2

        @pl.when(k >= 1)
        def _():
            drain_stores(k - 1)

        @pl.when(k + 1 < nmine)
        def _():
            drain_idx(k + 1)

            @pl.when(k + 2 < nmine)
            def _():
                fire_idx(k + 2)
            fire_gathers(k + 1)

        drain_gathers(k)
        for j in range(NSUB2):
            @plsc.parallel_loop(0, SUB // L, unroll=4)
            def grp(kg, j=j):
                sl = pl.ds(kg * L, L)
                ag = ag_v[p, j, sl]
                sm = s0_v[p, j, sl] + s1_v[p, j, sl]
                c_v[p, j, sl] = jnp.minimum(
                    jnp.maximum(ag * 5.0 / sm, 0.0), 1.0)

            pltpu.async_copy(c_v.at[p, j],
                             c_h.at[pl.ds(ebase + j * SUB, SUB)], sem_s)
        return carry

    lax.fori_loop(0, nmine, chunk_body, 0)
    drain_stores(nmine - 1)


_norm_kernel = functools.partial(
    pl.kernel,
    out_type=jax.ShapeDtypeStruct((N_EDGES,), jnp.float32),
    mesh=_mesh,
    scratch_types=[
        pltpu.VMEM((4, NSUB2, SUB), jnp.int32),
        pltpu.VMEM((2, NSUB2, SUB), jnp.float32),
        pltpu.VMEM((2, NSUB2, SUB), jnp.float32),
        pltpu.VMEM((2, NSUB2, SUB), jnp.float32),
        pltpu.VMEM((2, NSUB2, SUB), jnp.float32),
    ] + [pltpu.SemaphoreType.DMA] * 4,
    compiler_params=pltpu.CompilerParams(needs_layout_passes=False,
                                         use_tc_tiling_on_sc=False),
)(_norm_body)


def kernel(Eu, Ev, edge_index, g):
    src = edge_index[0]
    dst = edge_index[1]
    src2 = src.reshape(N_EDGES // SUB, SUB)
    dst2 = dst.reshape(N_EDGES // SUB, SUB)
    g2 = (g * g).reshape(D)
    dl = (jnp.arange(D)[:, None] + jnp.arange(L)[None, :]) % D
    g2b = g2[dl]
    zeros = jnp.zeros((N_NODES,), jnp.float32)
    ag, psum0, psum1 = _edge_kernel(src2, dst2, Eu, Ev, g2b, zeros)
    return _norm_kernel(src2, ag, psum0, psum1)


# speedup vs baseline: 2.8159x; 1.0013x over previous
import functools

import jax
import jax.numpy as jnp
from jax import lax
from jax.experimental import pallas as pl
from jax.experimental.pallas import tpu as pltpu
from jax.experimental.pallas import tpu_sc as plsc

N_NODES = 100000
N_EDGES = 1600000
D = 32
L = 16
NC = 2
NS = 16
NW = NC * NS
SUB = 128
NSUB = 5
CHUNK = SUB * NSUB
NCHUNKS = N_EDGES // CHUNK
BASE_CHUNKS = NCHUNKS // NW
EXTRA = NCHUNKS - BASE_CHUNKS * NW
DUMP = 6256
DUMP_LAST = N_NODES - 15 * DUMP
NSUB2 = 25
CHUNK2 = SUB * NSUB2
NCHUNKS2 = N_EDGES // CHUNK2
BASE2 = NCHUNKS2 // NW
EXTRA2 = NCHUNKS2 - BASE2 * NW

_mesh = plsc.VectorSubcoreMesh(core_axis_name="c", subcore_axis_name="s")


def _edge_body(src2_h, dst2_h, eu_h, ev_h, g2b_h, zeros_h,
               ag_h, psum0_h, psum1_h,
               src_v, dst_v, u_v, v_v, ag_v, g2_v, sums_s,
               sem_idx, *rest):
    gsems = (rest[0:NSUB], rest[NSUB:2 * NSUB])
    sem_s, sem_a = rest[2 * NSUB], rest[2 * NSUB + 1]
    cid = lax.axis_index("c")
    sid = lax.axis_index("s")
    wid = sid * NC + cid

    @pl.when(sid == 0)
    def _():
        pltpu.sync_copy(zeros_h, sums_s)

    pltpu.sync_copy(g2b_h, g2_v)
    plsc.subcore_barrier()

    nmine = jnp.where(wid < EXTRA, BASE_CHUNKS + 1, BASE_CHUNKS)

    def fire_idx(k):
        r = lax.rem(k, 4)
        row0 = (wid + k * NW) * NSUB
        pltpu.async_copy(src2_h.at[pl.ds(row0, NSUB)], src_v.at[r], sem_idx)
        pltpu.async_copy(dst2_h.at[pl.ds(row0, NSUB)], dst_v.at[r], sem_idx)

    def drain_idx(k):
        r = lax.rem(k, 4)
        pltpu.make_async_copy(src2_h.at[pl.ds(0, NSUB)], src_v.at[r],
                              sem_idx).wait()
        pltpu.make_async_copy(dst2_h.at[pl.ds(0, NSUB)], dst_v.at[r],
                              sem_idx).wait()

    def fire_gathers(k):
        r = lax.rem(k, 4)
        p = lax.rem(k, 2)
        for j in range(NSUB):
            for pp in range(2):
                @pl.when(p == pp)
                def _(pp=pp, j=j):
                    pltpu.async_copy(eu_h.at[src_v.at[r, j]], u_v.at[pp, j],
                                     gsems[pp][j])
                    pltpu.async_copy(ev_h.at[dst_v.at[r, j]], v_v.at[pp, j],
                                     gsems[pp][j])

    def drain_outs(k):
        r = lax.rem(k, 4)
        p = lax.rem(k, 2)
        chunk = wid + k * NW
        ebase = chunk * CHUNK
        for j in range(NSUB):
            pltpu.make_async_copy(
                ag_v.at[p, j], sums_s.at[src_v.at[r, j]], sem_a).wait()
            pltpu.make_async_copy(
                ag_v.at[p, j], ag_h.at[pl.ds(ebase + j * SUB, SUB)],
                sem_s).wait()

    fire_idx(0)
    drain_idx(0)

    @pl.when(1 < nmine)
    def _():
        fire_idx(1)
    fire_gathers(0)

    def chunk_body(k, carry):
        r = lax.rem(k, 4)
        p = lax.rem(k, 2)
        chunk = wid + k * NW
        ebase = chunk * CHUNK

        @pl.when(k >= 1)
        def _():
            drain_outs(k - 1)

        @pl.when(k + 1 < nmine)
        def _():
            drain_idx(k + 1)

            @pl.when(k + 2 < nmine)
            def _():
                fire_idx(k + 2)
            fire_gathers(k + 1)

        for j in range(NSUB):
            for pp in range(2):
                @pl.when(p == pp)
                def _(pp=pp, j=j):
                    pltpu.make_async_copy(eu_h.at[src_v.at[r, j]],
                                          u_v.at[pp, j], gsems[pp][j]).wait()
                    pltpu.make_async_copy(ev_h.at[dst_v.at[r, j]],
                                          v_v.at[pp, j], gsems[pp][j]).wait()

            @plsc.parallel_loop(0, SUB // L, unroll=4)
            def grp(kg, j=j):
                lane = lax.iota(jnp.int32, L)
                eidx = kg * L + lane
                acc = jnp.zeros((L,), jnp.float32)
                for d in range(D):
                    dsp = (d + lane) & (D - 1)
                    ud = plsc.load_gather(u_v.at[p, j], [eidx, dsp])
                    vd = plsc.load_gather(v_v.at[p, j], [eidx, dsp])
                    acc = acc + ud * vd * g2_v[d]
                t = jnp.exp(acc)
                ag = 1.0 / (1.0 + jnp.exp(-t))
                ag_v[p, j, pl.ds(kg * L, L)] = ag

            pltpu.async_copy(ag_v.at[p, j], sums_s.at[src_v.at[r, j]],
                             sem_a, add=True)
            pltpu.async_copy(ag_v.at[p, j],
                             ag_h.at[pl.ds(ebase + j * SUB, SUB)], sem_s)
        return carry

    lax.fori_loop(0, nmine, chunk_body, 0)
    drain_outs(nmine - 1)
    plsc.subcore_barrier()

    for c in range(NC):
        psum_h = (psum0_h, psum1_h)[c]

        @pl.when(jnp.logical_and(cid == c, sid < NS - 1))
        def _():
            pltpu.sync_copy(sums_s.at[pl.ds(sid * DUMP, DUMP)],
                            psum_h.at[pl.ds(sid * DUMP, DUMP)])

        @pl.when(jnp.logical_and(cid == c, sid == NS - 1))
        def _():
            pltpu.sync_copy(sums_s.at[pl.ds((NS - 1) * DUMP, DUMP_LAST)],
                            psum_h.at[pl.ds((NS - 1) * DUMP, DUMP_LAST)])


_edge_kernel = functools.partial(
    pl.kernel,
    out_type=(
        jax.ShapeDtypeStruct((N_EDGES,), jnp.float32),
        jax.ShapeDtypeStruct((N_NODES,), jnp.float32),
        jax.ShapeDtypeStruct((N_NODES,), jnp.float32),
    ),
    mesh=_mesh,
    scratch_types=[
        pltpu.VMEM((4, NSUB, SUB), jnp.int32),
        pltpu.VMEM((4, NSUB, SUB), jnp.int32),
        pltpu.VMEM((2, NSUB, SUB, D), jnp.float32),
        pltpu.VMEM((2, NSUB, SUB, D), jnp.float32),
        pltpu.VMEM((2, NSUB, SUB), jnp.float32),
        pltpu.VMEM((D, L), jnp.float32),
        pltpu.VMEM_SHARED((N_NODES,), jnp.float32),
    ] + [pltpu.SemaphoreType.DMA] * (2 * NSUB + 3),
    compiler_params=pltpu.CompilerParams(needs_layout_passes=False,
                                         use_tc_tiling_on_sc=False),
)(_edge_body)


def _norm_body(src2_h, ag_h, p0_h, p1_h,
               c_h,
               src_v, s0_v, s1_v, ag_v, c_v,
               sem_idx, sem_ga, sem_gb, sem_s):
    cid = lax.axis_index("c")
    sid = lax.axis_index("s")
    wid = sid * NC + cid
    gsems = (sem_ga, sem_gb)
    nmine = jnp.where(wid < EXTRA2, BASE2 + 1, BASE2)

    def fire_idx(k):
        r = lax.rem(k, 4)
        row0 = (wid + k * NW) * NSUB2
        pltpu.async_copy(src2_h.at[pl.ds(row0, NSUB2)], src_v.at[r], sem_idx)

    def drain_idx(k):
        r = lax.rem(k, 4)
        pltpu.make_async_copy(src2_h.at[pl.ds(0, NSUB2)], src_v.at[r],
                              sem_idx).wait()

    def transfers(k, p):
        r = lax.rem(k, 4)
        ebase = (wid + k * NW) * CHUNK2
        out = []
        for j in range(NSUB2):
            out.append(pltpu.make_async_copy(
                p0_h.at[src_v.at[r, j]], s0_v.at[p, j], gsems[p]))
            out.append(pltpu.make_async_copy(
                p1_h.at[src_v.at[r, j]], s1_v.at[p, j], gsems[p]))
            out.append(pltpu.make_async_copy(
                ag_h.at[pl.ds(ebase + j * SUB, SUB)], ag_v.at[p, j],
                gsems[p]))
        return out

    def fire_gathers(k):
        for pp in range(2):
            @pl.when(lax.rem(k, 2) == pp)
            def _(pp=pp):
                for c in transfers(k, pp):
                    c.start()

    def drain_gathers(k):
        for pp in range(2):
            @pl.when(lax.rem(k, 2) == pp)
            def _(pp=pp):
                for c in transfers(k, pp):
                    c.wait()

    def drain_stores(k):
        p = lax.rem(k, 2)
        ebase = (wid + k * NW) * CHUNK2
        for j in range(NSUB2):
            pltpu.make_async_copy(
                c_v.at[p, j], c_h.at[pl.ds(ebase + j * SUB, SUB)],
                sem_s).wait()

    fire_idx(0)
    drain_idx(0)

    @pl.when(1 < nmine)
    def _():
        fire_idx(1)
    fire_gathers(0)

    def chunk_body(k, carry):
        p = lax.rem(k, 2)
        ebase = (wid + k * NW) * CHUNK2

        @pl.when(k >= 1)
        def _():
            drain_stores(k - 1)

        @pl.when(k + 1 < nmine)
        def _():
            drain_idx(k + 1)

            @pl.when(k + 2 < nmine)
            def _():
                fire_idx(k + 2)
            fire_gathers(k + 1)

        drain_gathers(k)
        for j in range(NSUB2):
            @plsc.parallel_loop(0, SUB // L, unroll=4)
            def grp(kg, j=j):
                sl = pl.ds(kg * L, L)
                ag = ag_v[p, j, sl]
                sm = s0_v[p, j, sl] + s1_v[p, j, sl]
                c_v[p, j, sl] = jnp.minimum(
                    jnp.maximum(ag * 5.0 / sm, 0.0), 1.0)

            pltpu.async_copy(c_v.at[p, j],
                             c_h.at[pl.ds(ebase + j * SUB, SUB)], sem_s)
        return carry

    lax.fori_loop(0, nmine, chunk_body, 0)
    drain_stores(nmine - 1)


_norm_kernel = functools.partial(
    pl.kernel,
    out_type=jax.ShapeDtypeStruct((N_EDGES,), jnp.float32),
    mesh=_mesh,
    scratch_types=[
        pltpu.VMEM((4, NSUB2, SUB), jnp.int32),
        pltpu.VMEM((2, NSUB2, SUB), jnp.float32),
        pltpu.VMEM((2, NSUB2, SUB), jnp.float32),
        pltpu.VMEM((2, NSUB2, SUB), jnp.float32),
        pltpu.VMEM((2, NSUB2, SUB), jnp.float32),
    ] + [pltpu.SemaphoreType.DMA] * 4,
    compiler_params=pltpu.CompilerParams(needs_layout_passes=False,
                                         use_tc_tiling_on_sc=False),
)(_norm_body)


def kernel(Eu, Ev, edge_index, g):
    src = edge_index[0]
    dst = edge_index[1]
    src2 = src.reshape(N_EDGES // SUB, SUB)
    dst2 = dst.reshape(N_EDGES // SUB, SUB)
    g2 = (g * g).reshape(D)
    dl = (jnp.arange(D)[:, None] + jnp.arange(L)[None, :]) % D
    g2b = g2[dl]
    zeros = jnp.zeros((N_NODES,), jnp.float32)
    ag, psum0, psum1 = _edge_kernel(src2, dst2, Eu, Ev, g2b, zeros)
    return _norm_kernel(src2, ag, psum0, psum1)


# speedup vs baseline: 2.9151x; 1.0352x over previous
import functools

import jax
import jax.numpy as jnp
from jax import lax
from jax.experimental import pallas as pl
from jax.experimental.pallas import tpu as pltpu
from jax.experimental.pallas import tpu_sc as plsc

N_NODES = 100000
N_EDGES = 1600000
D = 32
L = 16
NC = 2
NS = 16
NW = NC * NS
SUB = 128
NSUB = 5
CHUNK = SUB * NSUB
NCHUNKS = N_EDGES // CHUNK
BASE_CHUNKS = NCHUNKS // NW
EXTRA = NCHUNKS - BASE_CHUNKS * NW
DUMP = 6256
DUMP_LAST = N_NODES - 15 * DUMP
NSUB2 = 25
CHUNK2 = SUB * NSUB2
NCHUNKS2 = N_EDGES // CHUNK2
BASE2 = NCHUNKS2 // NW
EXTRA2 = NCHUNKS2 - BASE2 * NW

_mesh = plsc.VectorSubcoreMesh(core_axis_name="c", subcore_axis_name="s")


def _edge_body(ei_h, eu_h, ev_h, g2b_h, zeros_h,
               ag_h, psum0_h, psum1_h,
               src_v, dst_v, u_v, v_v, ag_v, g2_v, sums_s,
               sem_idx, *rest):
    gsems = (rest[0:NSUB], rest[NSUB:2 * NSUB])
    sem_s, sem_a = rest[2 * NSUB], rest[2 * NSUB + 1]
    cid = lax.axis_index("c")
    sid = lax.axis_index("s")
    wid = sid * NC + cid

    @pl.when(sid == 0)
    def _():
        pltpu.sync_copy(zeros_h, sums_s)

    pltpu.sync_copy(g2b_h, g2_v)
    plsc.subcore_barrier()

    nmine = jnp.where(wid < EXTRA, BASE_CHUNKS + 1, BASE_CHUNKS)

    def fire_idx(k):
        r = lax.rem(k, 4)
        row0 = (wid + k * NW) * NSUB
        pltpu.async_copy(ei_h.at[0, pl.ds(row0, NSUB)], src_v.at[r], sem_idx)
        pltpu.async_copy(ei_h.at[1, pl.ds(row0, NSUB)], dst_v.at[r], sem_idx)

    def drain_idx(k):
        r = lax.rem(k, 4)
        pltpu.make_async_copy(ei_h.at[0, pl.ds(0, NSUB)], src_v.at[r],
                              sem_idx).wait()
        pltpu.make_async_copy(ei_h.at[1, pl.ds(0, NSUB)], dst_v.at[r],
                              sem_idx).wait()

    def fire_gathers(k):
        r = lax.rem(k, 4)
        p = lax.rem(k, 2)
        for j in range(NSUB):
            for pp in range(2):
                @pl.when(p == pp)
                def _(pp=pp, j=j):
                    pltpu.async_copy(eu_h.at[src_v.at[r, j]], u_v.at[pp, j],
                                     gsems[pp][j])
                    pltpu.async_copy(ev_h.at[dst_v.at[r, j]], v_v.at[pp, j],
                                     gsems[pp][j])

    def drain_outs(k):
        r = lax.rem(k, 4)
        p = lax.rem(k, 2)
        chunk = wid + k * NW
        ebase = chunk * CHUNK
        for j in range(NSUB):
            pltpu.make_async_copy(
                ag_v.at[p, j], sums_s.at[src_v.at[r, j]], sem_a).wait()
            pltpu.make_async_copy(
                ag_v.at[p, j], ag_h.at[pl.ds(ebase + j * SUB, SUB)],
                sem_s).wait()

    fire_idx(0)
    drain_idx(0)

    @pl.when(1 < nmine)
    def _():
        fire_idx(1)
    fire_gathers(0)

    def chunk_body(k, carry):
        r = lax.rem(k, 4)
        p = lax.rem(k, 2)
        chunk = wid + k * NW
        ebase = chunk * CHUNK

        @pl.when(k >= 1)
        def _():
            drain_outs(k - 1)

        @pl.when(k + 1 < nmine)
        def _():
            drain_idx(k + 1)

            @pl.when(k + 2 < nmine)
            def _():
                fire_idx(k + 2)
            fire_gathers(k + 1)

        for j in range(NSUB):
            for pp in range(2):
                @pl.when(p == pp)
                def _(pp=pp, j=j):
                    pltpu.make_async_copy(eu_h.at[src_v.at[r, j]],
                                          u_v.at[pp, j], gsems[pp][j]).wait()
                    pltpu.make_async_copy(ev_h.at[dst_v.at[r, j]],
                                          v_v.at[pp, j], gsems[pp][j]).wait()

            @plsc.parallel_loop(0, SUB // L, unroll=4)
            def grp(kg, j=j):
                lane = lax.iota(jnp.int32, L)
                eidx = kg * L + lane
                acc = jnp.zeros((L,), jnp.float32)
                for d in range(D):
                    dsp = (d + lane) & (D - 1)
                    ud = plsc.load_gather(u_v.at[p, j], [eidx, dsp])
                    vd = plsc.load_gather(v_v.at[p, j], [eidx, dsp])
                    acc = acc + ud * vd * g2_v[d]
                t = jnp.exp(acc)
                ag = 1.0 / (1.0 + jnp.exp(-t))
                ag_v[p, j, pl.ds(kg * L, L)] = ag

            pltpu.async_copy(ag_v.at[p, j], sums_s.at[src_v.at[r, j]],
                             sem_a, add=True)
            pltpu.async_copy(ag_v.at[p, j],
                             ag_h.at[pl.ds(ebase + j * SUB, SUB)], sem_s)
        return carry

    lax.fori_loop(0, nmine, chunk_body, 0)
    drain_outs(nmine - 1)
    plsc.subcore_barrier()

    for c in range(NC):
        psum_h = (psum0_h, psum1_h)[c]

        @pl.when(jnp.logical_and(cid == c, sid < NS - 1))
        def _():
            pltpu.sync_copy(sums_s.at[pl.ds(sid * DUMP, DUMP)],
                            psum_h.at[pl.ds(sid * DUMP, DUMP)])

        @pl.when(jnp.logical_and(cid == c, sid == NS - 1))
        def _():
            pltpu.sync_copy(sums_s.at[pl.ds((NS - 1) * DUMP, DUMP_LAST)],
                            psum_h.at[pl.ds((NS - 1) * DUMP, DUMP_LAST)])


_edge_kernel = functools.partial(
    pl.kernel,
    out_type=(
        jax.ShapeDtypeStruct((N_EDGES,), jnp.float32),
        jax.ShapeDtypeStruct((N_NODES,), jnp.float32),
        jax.ShapeDtypeStruct((N_NODES,), jnp.float32),
    ),
    mesh=_mesh,
    scratch_types=[
        pltpu.VMEM((4, NSUB, SUB), jnp.int32),
        pltpu.VMEM((4, NSUB, SUB), jnp.int32),
        pltpu.VMEM((2, NSUB, SUB, D), jnp.float32),
        pltpu.VMEM((2, NSUB, SUB, D), jnp.float32),
        pltpu.VMEM((2, NSUB, SUB), jnp.float32),
        pltpu.VMEM((D, L), jnp.float32),
        pltpu.VMEM_SHARED((N_NODES,), jnp.float32),
    ] + [pltpu.SemaphoreType.DMA] * (2 * NSUB + 3),
    compiler_params=pltpu.CompilerParams(needs_layout_passes=False,
                                         use_tc_tiling_on_sc=False),
)(_edge_body)


def _norm_body(ei_h, ag_h, p0_h, p1_h,
               c_h,
               src_v, s0_v, s1_v, ag_v, c_v,
               sem_idx, sem_ga, sem_gb, sem_s):
    cid = lax.axis_index("c")
    sid = lax.axis_index("s")
    wid = sid * NC + cid
    gsems = (sem_ga, sem_gb)
    nmine = jnp.where(wid < EXTRA2, BASE2 + 1, BASE2)

    def fire_idx(k):
        r = lax.rem(k, 4)
        row0 = (wid + k * NW) * NSUB2
        pltpu.async_copy(ei_h.at[0, pl.ds(row0, NSUB2)], src_v.at[r],
                         sem_idx)

    def drain_idx(k):
        r = lax.rem(k, 4)
        pltpu.make_async_copy(ei_h.at[0, pl.ds(0, NSUB2)], src_v.at[r],
                              sem_idx).wait()

    def transfers(k, p):
        r = lax.rem(k, 4)
        ebase = (wid + k * NW) * CHUNK2
        out = []
        for j in range(NSUB2):
            out.append(pltpu.make_async_copy(
                p0_h.at[src_v.at[r, j]], s0_v.at[p, j], gsems[p]))
            out.append(pltpu.make_async_copy(
                p1_h.at[src_v.at[r, j]], s1_v.at[p, j], gsems[p]))
            out.append(pltpu.make_async_copy(
                ag_h.at[pl.ds(ebase + j * SUB, SUB)], ag_v.at[p, j],
                gsems[p]))
        return out

    def fire_gathers(k):
        for pp in range(2):
            @pl.when(lax.rem(k, 2) == pp)
            def _(pp=pp):
                for c in transfers(k, pp):
                    c.start()

    def drain_gathers(k):
        for pp in range(2):
            @pl.when(lax.rem(k, 2) == pp)
            def _(pp=pp):
                for c in transfers(k, pp):
                    c.wait()

    def drain_stores(k):
        p = lax.rem(k, 2)
        ebase = (wid + k * NW) * CHUNK2
        for j in range(NSUB2):
            pltpu.make_async_copy(
                c_v.at[p, j], c_h.at[pl.ds(ebase + j * SUB, SUB)],
                sem_s).wait()

    fire_idx(0)
    drain_idx(0)

    @pl.when(1 < nmine)
    def _():
        fire_idx(1)
    fire_gathers(0)

    def chunk_body(k, carry):
        p = lax.rem(k, 2)
        ebase = (wid + k * NW) * CHUNK2

        @pl.when(k >= 1)
        def _():
            drain_stores(k - 1)

        @pl.when(k + 1 < nmine)
        def _():
            drain_idx(k + 1)

            @pl.when(k + 2 < nmine)
            def _():
                fire_idx(k + 2)
            fire_gathers(k + 1)

        drain_gathers(k)
        for j in range(NSUB2):
            @plsc.parallel_loop(0, SUB // L, unroll=4)
            def grp(kg, j=j):
                sl = pl.ds(kg * L, L)
                ag = ag_v[p, j, sl]
                sm = s0_v[p, j, sl] + s1_v[p, j, sl]
                c_v[p, j, sl] = jnp.minimum(
                    jnp.maximum(ag * 5.0 / sm, 0.0), 1.0)

            pltpu.async_copy(c_v.at[p, j],
                             c_h.at[pl.ds(ebase + j * SUB, SUB)], sem_s)
        return carry

    lax.fori_loop(0, nmine, chunk_body, 0)
    drain_stores(nmine - 1)


_norm_kernel = functools.partial(
    pl.kernel,
    out_type=jax.ShapeDtypeStruct((N_EDGES,), jnp.float32),
    mesh=_mesh,
    scratch_types=[
        pltpu.VMEM((4, NSUB2, SUB), jnp.int32),
        pltpu.VMEM((2, NSUB2, SUB), jnp.float32),
        pltpu.VMEM((2, NSUB2, SUB), jnp.float32),
        pltpu.VMEM((2, NSUB2, SUB), jnp.float32),
        pltpu.VMEM((2, NSUB2, SUB), jnp.float32),
    ] + [pltpu.SemaphoreType.DMA] * 4,
    compiler_params=pltpu.CompilerParams(needs_layout_passes=False,
                                         use_tc_tiling_on_sc=False),
)(_norm_body)


def kernel(Eu, Ev, edge_index, g):
    ei3 = edge_index.reshape(2, N_EDGES // SUB, SUB)
    g2 = (g * g).reshape(D)
    dl = (jnp.arange(D)[:, None] + jnp.arange(L)[None, :]) % D
    g2b = g2[dl]
    zeros = jnp.zeros((N_NODES,), jnp.float32)
    ag, psum0, psum1 = _edge_kernel(ei3, Eu, Ev, g2b, zeros)
    return _norm_kernel(ei3, ag, psum0, psum1)
